# baseline (speedup 1.0000x reference)
import functools

import jax
import jax.numpy as jnp
from jax import lax
from jax.experimental import pallas as pl
from jax.experimental.pallas import tpu as pltpu
from jax.experimental.pallas import tpu_sc as plsc

NC = 2
NS = 16
NW = NC * NS
CK = 128


def _sc_gather(tab0, tab1, e0p, e1p, ne):
    nn, h = tab0.shape
    _, nchp, _ = e0p.shape
    ewp = nchp * CK
    mesh = plsc.VectorSubcoreMesh(core_axis_name="c", subcore_axis_name="s")

    @functools.partial(
        pl.kernel,
        out_type=(jax.ShapeDtypeStruct((ne, h), jnp.float32),
                  jax.ShapeDtypeStruct((ne, h), jnp.float32)),
        mesh=mesh,
        scratch_types=[
            pltpu.VMEM((nchp, CK), jnp.int32),
            pltpu.VMEM((nchp, CK), jnp.int32),
            pltpu.VMEM((CK, h), jnp.float32),
            pltpu.VMEM((CK, h), jnp.float32),
            pltpu.VMEM((CK, h), jnp.float32),
            pltpu.VMEM((CK, h), jnp.float32),
        ] + [pltpu.SemaphoreType.DMA] * 8,
    )
    def gk(t0, t1, e0h, e1h, o0, o1, idx0, idx1, r0a, r1a, r0b, r1b,
           sg0a, sg1a, sg0b, sg1b, so0a, so1a, so0b, so1b):
        wid = lax.axis_index("c") * NS + lax.axis_index("s")
        base = wid * ewp
        nch = jnp.minimum(ewp, ne - base) // CK
        pltpu.sync_copy(e0h.at[wid], idx0)
        pltpu.sync_copy(e1h.at[wid], idx1)

        def issue_g(j, r0, r1, s0, s1):
            pltpu.async_copy(t0.at[idx0.at[j]], r0, s0)
            pltpu.async_copy(t1.at[idx1.at[j]], r1, s1)

        def wait_g(j, r0, r1, s0, s1):
            pltpu.make_async_copy(t0.at[idx0.at[j]], r0, s0).wait()
            pltpu.make_async_copy(t1.at[idx1.at[j]], r1, s1).wait()

        def issue_o(j, r0, r1, s0, s1):
            off = pl.multiple_of(base + j * CK, 8)
            pltpu.async_copy(r0, o0.at[pl.ds(off, CK)], s0)
            pltpu.async_copy(r1, o1.at[pl.ds(off, CK)], s1)

        def wait_o(j, r0, r1, s0, s1):
            off = pl.multiple_of(base + j * CK, 8)
            pltpu.make_async_copy(r0, o0.at[pl.ds(off, CK)], s0).wait()
            pltpu.make_async_copy(r1, o1.at[pl.ds(off, CK)], s1).wait()

        issue_g(0, r0a, r1a, sg0a, sg1a)
        issue_g(1, r0b, r1b, sg0b, sg1b)

        def body(i, carry):
            j = 2 * i
            wait_g(j, r0a, r1a, sg0a, sg1a)
            issue_o(j, r0a, r1a, so0a, so1a)

            @pl.when(j + 1 < nch)
            def _():
                wait_g(j + 1, r0b, r1b, sg0b, sg1b)
                issue_o(j + 1, r0b, r1b, so0b, so1b)

            @pl.when(j + 2 < nch)
            def _():
                wait_o(j, r0a, r1a, so0a, so1a)
                issue_g(j + 2, r0a, r1a, sg0a, sg1a)

            @pl.when(j + 3 < nch)
            def _():
                wait_o(j + 1, r0b, r1b, so0b, so1b)
                issue_g(j + 3, r0b, r1b, sg0b, sg1b)

            return carry

        lax.fori_loop(0, (nch + 1) // 2, body, 0)
        par = nch % 2
        wait_o(nch - 2 + par, r0a, r1a, so0a, so1a)
        wait_o(nch - 1 - par, r0b, r1b, so0b, so1b)

    return gk(tab0, tab1, e0p, e1p)


def _sc_scatter(x_edge_new, e0p, e1p, zeros):
    ne, h = x_edge_new.shape
    nn = zeros.shape[0]
    _, nchp, _ = e0p.shape
    ewp = nchp * CK
    npt = (nn // NS) // 8 * 8
    rem = nn - NS * npt
    mesh = plsc.VectorSubcoreMesh(core_axis_name="c", subcore_axis_name="s")

    @functools.partial(
        pl.kernel,
        out_type=jax.ShapeDtypeStruct((NC, nn, h), jnp.float32),
        mesh=mesh,
        scratch_types=[
            pltpu.VMEM_SHARED((nn, h), jnp.float32),
            pltpu.VMEM((1, CK), jnp.int32),
            pltpu.VMEM((1, CK), jnp.int32),
            pltpu.VMEM((1, CK), jnp.int32),
            pltpu.VMEM((1, CK), jnp.int32),
            pltpu.VMEM((CK, h), jnp.float32),
            pltpu.VMEM((CK, h), jnp.float32),
        ] + [pltpu.SemaphoreType.DMA] * 10,
    )
    def sk(xen, e0h, e1h, zz, oout, agg_sh, i0a, i1a, i0b, i1b, ra, rb,
           sla, slb, si0a, si1a, si0b, si1b, s0a, s1a, s0b, s1b):
        cid = lax.axis_index("c")
        sid = lax.axis_index("s")
        wid = cid * NS + sid
        base = wid * ewp
        nch = jnp.minimum(ewp, ne - base) // CK
        off_n = pl.multiple_of(sid * npt, 8)
        pltpu.sync_copy(zz.at[pl.ds(off_n, npt)],
                        agg_sh.at[pl.ds(off_n, npt)])
        if rem:
            @pl.when(sid == NS - 1)
            def _():
                pltpu.sync_copy(zz.at[pl.ds(NS * npt, rem)],
                                agg_sh.at[pl.ds(NS * npt, rem)])
        plsc.subcore_barrier()

        def issue_l(j, r, i0, i1, sr, s0, s1):
            off = pl.multiple_of(base + j * CK, 8)
            pltpu.async_copy(xen.at[pl.ds(off, CK)], r, sr)
            pltpu.async_copy(e0h.at[wid, j], i0.at[0], s0)
            pltpu.async_copy(e1h.at[wid, j], i1.at[0], s1)

        def wait_l(j, r, i0, i1, sr, s0, s1):
            off = pl.multiple_of(base + j * CK, 8)
            pltpu.make_async_copy(xen.at[pl.ds(off, CK)], r, sr).wait()
            pltpu.make_async_copy(e0h.at[wid, j], i0.at[0], s0).wait()
            pltpu.make_async_copy(e1h.at[wid, j], i1.at[0], s1).wait()

        def issue_s(r, i0, i1, s0, s1):
            pltpu.async_copy(r, agg_sh.at[i0.at[0]], s0, add=True)
            pltpu.async_copy(r, agg_sh.at[i1.at[0]], s1, add=True)

        def wait_s(r, i0, i1, s0, s1):
            pltpu.make_async_copy(r, agg_sh.at[i0.at[0]], s0).wait()
            pltpu.make_async_copy(r, agg_sh.at[i1.at[0]], s1).wait()

        issue_l(0, ra, i0a, i1a, sla, si0a, si1a)
        issue_l(1, rb, i0b, i1b, slb, si0b, si1b)

        def body(i, carry):
            j = 2 * i
            wait_l(j, ra, i0a, i1a, sla, si0a, si1a)
            issue_s(ra, i0a, i1a, s0a, s1a)

            @pl.when(j + 1 < nch)
            def _():
                wait_l(j + 1, rb, i0b, i1b, slb, si0b, si1b)
                issue_s(rb, i0b, i1b, s0b, s1b)

            @pl.when(j + 2 < nch)
            def _():
                wait_s(ra, i0a, i1a, s0a, s1a)
                issue_l(j + 2, ra, i0a, i1a, sla, si0a, si1a)

            @pl.when(j + 3 < nch)
            def _():
                wait_s(rb, i0b, i1b, s0b, s1b)
                issue_l(j + 3, rb, i0b, i1b, slb, si0b, si1b)

            return carry

        lax.fori_loop(0, (nch + 1) // 2, body, 0)
        wait_s(ra, i0a, i1a, s0a, s1a)
        wait_s(rb, i0b, i1b, s0b, s1b)
        plsc.subcore_barrier()
        pltpu.sync_copy(agg_sh.at[pl.ds(off_n, npt)],
                        oout.at[cid, pl.ds(off_n, npt)])
        if rem:
            @pl.when(sid == NS - 1)
            def _():
                pltpu.sync_copy(agg_sh.at[pl.ds(NS * npt, rem)],
                                oout.at[cid, pl.ds(NS * npt, rem)])

    return sk(x_edge_new, e0p, e1p, zeros)


def _bdot(x, w):
    return jnp.dot(x.astype(jnp.bfloat16), w.astype(jnp.bfloat16),
                   preferred_element_type=jnp.float32)


def _mlp_tail(x, w2, b2, w3, b3, w4, b4, g, b):
    x = jnp.maximum(_bdot(x, w2) + b2, 0.0)
    x = jnp.maximum(_bdot(x, w3) + b3, 0.0)
    x = _bdot(x, w4) + b4
    mu = jnp.mean(x, axis=-1, keepdims=True)
    xc = x - mu
    var = jnp.mean(xc * xc, axis=-1, keepdims=True)
    return xc * lax.rsqrt(var + 1e-5) * g + b


def _pq(x_node, w1a, w1b):
    nn, h = x_node.shape
    blk = 1000
    nspec = pl.BlockSpec((blk, h), lambda i: (i, 0))
    wspec = pl.BlockSpec((h, h), lambda i: (0, 0))

    def body(xr, war, wbr, pr, qr):
        pr[...] = jnp.dot(xr[...], war[...], preferred_element_type=jnp.float32)
        qr[...] = jnp.dot(xr[...], wbr[...], preferred_element_type=jnp.float32)

    return pl.pallas_call(
        body,
        grid=(nn // blk,),
        in_specs=[nspec, wspec, wspec],
        out_specs=[nspec, nspec],
        out_shape=[jax.ShapeDtypeStruct((nn, h), jnp.float32)] * 2,
    )(x_node, w1a, w1b)


def _edge_mlp(g0, g1, xe, w1c, b1, w2, b2, w3, b3, w4, b4, g, b):
    ne, h = xe.shape
    blk = 2560
    grid = ne // blk
    espec = pl.BlockSpec((blk, h), lambda i: (i, 0))
    wspec = pl.BlockSpec((h, h), lambda i: (0, 0))
    vspec = pl.BlockSpec((1, h), lambda i: (0, 0))

    def body(g0r, g1r, xer, w1cr, b1r, w2r, b2r, w3r, b3r,
             w4r, b4r, gr, br, xnr, oer):
        x = g0r[...] + g1r[...] + _bdot(xer[...], w1cr[...]) + b1r[...]
        x = jnp.maximum(x, 0.0)
        xn = _mlp_tail(x, w2r[...], b2r[...], w3r[...], b3r[...],
                       w4r[...], b4r[...], gr[...], br[...])
        xnr[...] = xn
        oer[...] = xer[...] + xn

    return pl.pallas_call(
        body,
        grid=(grid,),
        in_specs=[espec, espec, espec, wspec, vspec, wspec,
                  vspec, wspec, vspec, wspec, vspec, vspec, vspec],
        out_specs=[espec, espec],
        out_shape=[jax.ShapeDtypeStruct((ne, h), jnp.float32)] * 2,
    )(g0, g1, xe, w1c, b1, w2, b2, w3, b3, w4, b4, g, b)


def _node_mlp(x_node, aggp, u1a, u1b, c1, u2, c2, u3, c3, u4, c4, g, b):
    nn, h = x_node.shape
    blk = 1000
    grid = nn // blk
    nspec = pl.BlockSpec((blk, h), lambda i: (i, 0))
    aspec = pl.BlockSpec((NC, blk, h), lambda i: (0, i, 0))
    wspec = pl.BlockSpec((h, h), lambda i: (0, 0))
    vspec = pl.BlockSpec((1, h), lambda i: (0, 0))

    def body(xnr, apr, u1ar, u1br, c1r, u2r, c2r, u3r, c3r, u4r, c4r,
             gr, br, outr):
        agg = apr[0] + apr[1]
        x = _bdot(xnr[...], u1ar[...]) + _bdot(agg, u1br[...]) + c1r[...]
        x = jnp.maximum(x, 0.0)
        xn = _mlp_tail(x, u2r[...], c2r[...], u3r[...], c3r[...],
                       u4r[...], c4r[...], gr[...], br[...])
        outr[...] = xnr[...] + xn

    return pl.pallas_call(
        body,
        grid=(grid,),
        in_specs=[nspec, aspec, wspec, wspec, vspec, wspec, vspec, wspec,
                  vspec, wspec, vspec, vspec, vspec],
        out_specs=nspec,
        out_shape=jax.ShapeDtypeStruct((nn, h), jnp.float32),
    )(x_node, aggp, u1a, u1b, c1, u2, c2, u3, c3, u4, c4, g, b)


def kernel(x_node, x_edge, edge_index, eb_params, nb_params):
    eb_layers, (eg, ebb) = eb_params
    nb_layers, (ng, nbb) = nb_params
    (w1, b1), (w2, b2), (w3, b3), (w4, b4) = eb_layers
    (u1, c1), (u2, c2), (u3, c3), (u4, c4) = nb_layers
    nn, h = x_node.shape
    r2 = lambda a: a.reshape(1, h)

    ne = edge_index.shape[0]
    nchp = -(-ne // (NW * CK))
    pad = NW * nchp * CK - ne
    e0p = jnp.pad(edge_index[:, 0].astype(jnp.int32), (0, pad)).reshape(NW, nchp, CK)
    e1p = jnp.pad(edge_index[:, 1].astype(jnp.int32), (0, pad)).reshape(NW, nchp, CK)

    p, q = _pq(x_node, w1[:h], w1[h:2 * h])
    g0, g1 = _sc_gather(p, q, e0p, e1p, ne)
    xen, out_edge = _edge_mlp(
        g0, g1, x_edge, w1[2 * h:], r2(b1),
        w2, r2(b2), w3, r2(b3), w4, r2(b4), r2(eg), r2(ebb))
    zeros = jnp.zeros((nn, h), jnp.float32)
    aggp = _sc_scatter(xen, e0p, e1p, zeros)
    out_node = _node_mlp(
        x_node, aggp, u1[:h], u1[h:], r2(c1),
        u2, r2(c2), u3, r2(c3), u4, r2(c4), r2(ng), r2(nbb))
    return (out_node, out_edge)


# speedup vs baseline: 1.0808x; 1.0808x over previous
import functools

import jax
import jax.numpy as jnp
from jax import lax
from jax.experimental import pallas as pl
from jax.experimental.pallas import tpu as pltpu
from jax.experimental.pallas import tpu_sc as plsc

NC = 2
NS = 16
NW = NC * NS
CK = 128


def _sc_gather_add(tab0, tab1, e0p, e1p, ne):
    nn, h = tab0.shape
    _, nchp, _ = e0p.shape
    ewp = nchp * CK
    nvec = h // 16
    mesh = plsc.VectorSubcoreMesh(core_axis_name="c", subcore_axis_name="s")

    @functools.partial(
        pl.kernel,
        out_type=jax.ShapeDtypeStruct((ne, h), jnp.float32),
        mesh=mesh,
        scratch_types=[
            pltpu.VMEM((nchp, CK), jnp.int32),
            pltpu.VMEM((nchp, CK), jnp.int32),
            pltpu.VMEM((CK, h), jnp.float32),
            pltpu.VMEM((CK, h), jnp.float32),
            pltpu.VMEM((CK, h), jnp.float32),
            pltpu.VMEM((CK, h), jnp.float32),
        ] + [pltpu.SemaphoreType.DMA] * 6,
    )
    def gk(t0, t1, e0h, e1h, og, idx0, idx1, r0a, r1a, r0b, r1b,
           sg0a, sg1a, sg0b, sg1b, soa, sob):
        wid = lax.axis_index("c") * NS + lax.axis_index("s")
        base = wid * ewp
        nch = jnp.minimum(ewp, ne - base) // CK
        pltpu.sync_copy(e0h.at[wid], idx0)
        pltpu.sync_copy(e1h.at[wid], idx1)

        def issue_g(j, r0, r1, s0, s1):
            pltpu.async_copy(t0.at[idx0.at[j]], r0, s0)
            pltpu.async_copy(t1.at[idx1.at[j]], r1, s1)

        def wait_g(j, r0, r1, s0, s1):
            pltpu.make_async_copy(t0.at[idx0.at[j]], r0, s0).wait()
            pltpu.make_async_copy(t1.at[idx1.at[j]], r1, s1).wait()

        def add_rows(r0, r1):
            def row(r, carry):
                for c in range(nvec):
                    sl = pl.ds(c * 16, 16)
                    r0[r, sl] = r0[r, sl] + r1[r, sl]
                return carry
            lax.fori_loop(0, CK, row, 0)

        def issue_o(j, r0, s):
            off = pl.multiple_of(base + j * CK, 8)
            pltpu.async_copy(r0, og.at[pl.ds(off, CK)], s)

        def wait_o(j, r0, s):
            off = pl.multiple_of(base + j * CK, 8)
            pltpu.make_async_copy(r0, og.at[pl.ds(off, CK)], s).wait()

        issue_g(0, r0a, r1a, sg0a, sg1a)
        issue_g(1, r0b, r1b, sg0b, sg1b)

        def body(i, carry):
            j = 2 * i
            wait_g(j, r0a, r1a, sg0a, sg1a)
            add_rows(r0a, r1a)
            issue_o(j, r0a, soa)

            @pl.when(j + 1 < nch)
            def _():
                wait_g(j + 1, r0b, r1b, sg0b, sg1b)
                add_rows(r0b, r1b)
                issue_o(j + 1, r0b, sob)

            @pl.when(j + 2 < nch)
            def _():
                wait_o(j, r0a, soa)
                issue_g(j + 2, r0a, r1a, sg0a, sg1a)

            @pl.when(j + 3 < nch)
            def _():
                wait_o(j + 1, r0b, sob)
                issue_g(j + 3, r0b, r1b, sg0b, sg1b)

            return carry

        lax.fori_loop(0, (nch + 1) // 2, body, 0)
        par = nch % 2
        wait_o(nch - 2 + par, r0a, soa)
        wait_o(nch - 1 - par, r0b, sob)

    return gk(tab0, tab1, e0p, e1p)


def _sc_scatter(x_edge_new, e0p, e1p, zeros):
    ne, h = x_edge_new.shape
    nn = zeros.shape[0]
    _, nchp, _ = e0p.shape
    ewp = nchp * CK
    npt = (nn // NS) // 8 * 8
    rem = nn - NS * npt
    mesh = plsc.VectorSubcoreMesh(core_axis_name="c", subcore_axis_name="s")

    @functools.partial(
        pl.kernel,
        out_type=jax.ShapeDtypeStruct((NC, nn, h), jnp.float32),
        mesh=mesh,
        scratch_types=[
            pltpu.VMEM_SHARED((nn, h), jnp.float32),
            pltpu.VMEM((1, CK), jnp.int32),
            pltpu.VMEM((1, CK), jnp.int32),
            pltpu.VMEM((1, CK), jnp.int32),
            pltpu.VMEM((1, CK), jnp.int32),
            pltpu.VMEM((CK, h), jnp.float32),
            pltpu.VMEM((CK, h), jnp.float32),
        ] + [pltpu.SemaphoreType.DMA] * 10,
    )
    def sk(xen, e0h, e1h, zz, oout, agg_sh, i0a, i1a, i0b, i1b, ra, rb,
           sla, slb, si0a, si1a, si0b, si1b, s0a, s1a, s0b, s1b):
        cid = lax.axis_index("c")
        sid = lax.axis_index("s")
        wid = cid * NS + sid
        base = wid * ewp
        nch = jnp.minimum(ewp, ne - base) // CK
        off_n = pl.multiple_of(sid * npt, 8)
        pltpu.sync_copy(zz.at[pl.ds(off_n, npt)],
                        agg_sh.at[pl.ds(off_n, npt)])
        if rem:
            @pl.when(sid == NS - 1)
            def _():
                pltpu.sync_copy(zz.at[pl.ds(NS * npt, rem)],
                                agg_sh.at[pl.ds(NS * npt, rem)])
        plsc.subcore_barrier()

        def issue_l(j, r, i0, i1, sr, s0, s1):
            off = pl.multiple_of(base + j * CK, 8)
            pltpu.async_copy(xen.at[pl.ds(off, CK)], r, sr)
            pltpu.async_copy(e0h.at[wid, j], i0.at[0], s0)
            pltpu.async_copy(e1h.at[wid, j], i1.at[0], s1)

        def wait_l(j, r, i0, i1, sr, s0, s1):
            off = pl.multiple_of(base + j * CK, 8)
            pltpu.make_async_copy(xen.at[pl.ds(off, CK)], r, sr).wait()
            pltpu.make_async_copy(e0h.at[wid, j], i0.at[0], s0).wait()
            pltpu.make_async_copy(e1h.at[wid, j], i1.at[0], s1).wait()

        def issue_s(r, i0, i1, s0, s1):
            pltpu.async_copy(r, agg_sh.at[i0.at[0]], s0, add=True)
            pltpu.async_copy(r, agg_sh.at[i1.at[0]], s1, add=True)

        def wait_s(r, i0, i1, s0, s1):
            pltpu.make_async_copy(r, agg_sh.at[i0.at[0]], s0).wait()
            pltpu.make_async_copy(r, agg_sh.at[i1.at[0]], s1).wait()

        issue_l(0, ra, i0a, i1a, sla, si0a, si1a)
        issue_l(1, rb, i0b, i1b, slb, si0b, si1b)

        def body(i, carry):
            j = 2 * i
            wait_l(j, ra, i0a, i1a, sla, si0a, si1a)
            issue_s(ra, i0a, i1a, s0a, s1a)

            @pl.when(j + 1 < nch)
            def _():
                wait_l(j + 1, rb, i0b, i1b, slb, si0b, si1b)
                issue_s(rb, i0b, i1b, s0b, s1b)

            @pl.when(j + 2 < nch)
            def _():
                wait_s(ra, i0a, i1a, s0a, s1a)
                issue_l(j + 2, ra, i0a, i1a, sla, si0a, si1a)

            @pl.when(j + 3 < nch)
            def _():
                wait_s(rb, i0b, i1b, s0b, s1b)
                issue_l(j + 3, rb, i0b, i1b, slb, si0b, si1b)

            return carry

        lax.fori_loop(0, (nch + 1) // 2, body, 0)
        wait_s(ra, i0a, i1a, s0a, s1a)
        wait_s(rb, i0b, i1b, s0b, s1b)
        plsc.subcore_barrier()
        pltpu.sync_copy(agg_sh.at[pl.ds(off_n, npt)],
                        oout.at[cid, pl.ds(off_n, npt)])
        if rem:
            @pl.when(sid == NS - 1)
            def _():
                pltpu.sync_copy(agg_sh.at[pl.ds(NS * npt, rem)],
                                oout.at[cid, pl.ds(NS * npt, rem)])

    return sk(x_edge_new, e0p, e1p, zeros)


def _bdot(x, w):
    return jnp.dot(x.astype(jnp.bfloat16), w.astype(jnp.bfloat16),
                   preferred_element_type=jnp.float32)


def _mlp_tail(x, w2, b2, w3, b3, w4, b4, g, b):
    x = jnp.maximum(_bdot(x, w2) + b2, 0.0)
    x = jnp.maximum(_bdot(x, w3) + b3, 0.0)
    x = _bdot(x, w4) + b4
    mu = jnp.mean(x, axis=-1, keepdims=True)
    xc = x - mu
    var = jnp.mean(xc * xc, axis=-1, keepdims=True)
    return xc * lax.rsqrt(var + 1e-5) * g + b


def _pq(x_node, w1a, w1b):
    nn, h = x_node.shape
    blk = 1000
    nspec = pl.BlockSpec((blk, h), lambda i: (i, 0))
    wspec = pl.BlockSpec((h, h), lambda i: (0, 0))

    def body(xr, war, wbr, pr, qr):
        pr[...] = jnp.dot(xr[...], war[...], preferred_element_type=jnp.float32)
        qr[...] = jnp.dot(xr[...], wbr[...], preferred_element_type=jnp.float32)

    return pl.pallas_call(
        body,
        grid=(nn // blk,),
        in_specs=[nspec, wspec, wspec],
        out_specs=[nspec, nspec],
        out_shape=[jax.ShapeDtypeStruct((nn, h), jnp.float32)] * 2,
    )(x_node, w1a, w1b)


def _edge_mlp(gsum, xe, w1c, b1, w2, b2, w3, b3, w4, b4, g, b):
    ne, h = xe.shape
    blk = 2560
    grid = ne // blk
    espec = pl.BlockSpec((blk, h), lambda i: (i, 0))
    wspec = pl.BlockSpec((h, h), lambda i: (0, 0))
    vspec = pl.BlockSpec((1, h), lambda i: (0, 0))

    def body(gsr, xer, w1cr, b1r, w2r, b2r, w3r, b3r,
             w4r, b4r, gr, br, xnr, oer):
        x = gsr[...] + _bdot(xer[...], w1cr[...]) + b1r[...]
        x = jnp.maximum(x, 0.0)
        xn = _mlp_tail(x, w2r[...], b2r[...], w3r[...], b3r[...],
                       w4r[...], b4r[...], gr[...], br[...])
        xnr[...] = xn
        oer[...] = xer[...] + xn

    return pl.pallas_call(
        body,
        grid=(grid,),
        in_specs=[espec, espec, wspec, vspec, wspec,
                  vspec, wspec, vspec, wspec, vspec, vspec, vspec],
        out_specs=[espec, espec],
        out_shape=[jax.ShapeDtypeStruct((ne, h), jnp.float32)] * 2,
    )(gsum, xe, w1c, b1, w2, b2, w3, b3, w4, b4, g, b)


def _node_mlp(x_node, aggp, u1a, u1b, c1, u2, c2, u3, c3, u4, c4, g, b):
    nn, h = x_node.shape
    blk = 1000
    grid = nn // blk
    nspec = pl.BlockSpec((blk, h), lambda i: (i, 0))
    aspec = pl.BlockSpec((NC, blk, h), lambda i: (0, i, 0))
    wspec = pl.BlockSpec((h, h), lambda i: (0, 0))
    vspec = pl.BlockSpec((1, h), lambda i: (0, 0))

    def body(xnr, apr, u1ar, u1br, c1r, u2r, c2r, u3r, c3r, u4r, c4r,
             gr, br, outr):
        agg = apr[0] + apr[1]
        x = _bdot(xnr[...], u1ar[...]) + _bdot(agg, u1br[...]) + c1r[...]
        x = jnp.maximum(x, 0.0)
        xn = _mlp_tail(x, u2r[...], c2r[...], u3r[...], c3r[...],
                       u4r[...], c4r[...], gr[...], br[...])
        outr[...] = xnr[...] + xn

    return pl.pallas_call(
        body,
        grid=(grid,),
        in_specs=[nspec, aspec, wspec, wspec, vspec, wspec, vspec, wspec,
                  vspec, wspec, vspec, vspec, vspec],
        out_specs=nspec,
        out_shape=jax.ShapeDtypeStruct((nn, h), jnp.float32),
    )(x_node, aggp, u1a, u1b, c1, u2, c2, u3, c3, u4, c4, g, b)


def kernel(x_node, x_edge, edge_index, eb_params, nb_params):
    eb_layers, (eg, ebb) = eb_params
    nb_layers, (ng, nbb) = nb_params
    (w1, b1), (w2, b2), (w3, b3), (w4, b4) = eb_layers
    (u1, c1), (u2, c2), (u3, c3), (u4, c4) = nb_layers
    nn, h = x_node.shape
    r2 = lambda a: a.reshape(1, h)

    ne = edge_index.shape[0]
    nchp = -(-ne // (NW * CK))
    pad = NW * nchp * CK - ne
    e0p = jnp.pad(edge_index[:, 0].astype(jnp.int32), (0, pad)).reshape(NW, nchp, CK)
    e1p = jnp.pad(edge_index[:, 1].astype(jnp.int32), (0, pad)).reshape(NW, nchp, CK)

    p, q = _pq(x_node, w1[:h], w1[h:2 * h])
    gsum = _sc_gather_add(p, q, e0p, e1p, ne)
    xen, out_edge = _edge_mlp(
        gsum, x_edge, w1[2 * h:], r2(b1),
        w2, r2(b2), w3, r2(b3), w4, r2(b4), r2(eg), r2(ebb))
    zeros = jnp.zeros((nn, h), jnp.float32)
    aggp = _sc_scatter(xen, e0p, e1p, zeros)
    out_node = _node_mlp(
        x_node, aggp, u1[:h], u1[h:], r2(c1),
        u2, r2(c2), u3, r2(c3), u4, r2(c4), r2(ng), r2(nbb))
    return (out_node, out_edge)


# speedup vs baseline: 1.0816x; 1.0008x over previous
import functools

import jax
import jax.numpy as jnp
from jax import lax
from jax.experimental import pallas as pl
from jax.experimental.pallas import tpu as pltpu
from jax.experimental.pallas import tpu_sc as plsc

NC = 2
NS = 16
NW = NC * NS
CK = 128


def _sc_gather_add(tab0, tab1, e0p, e1p, ne):
    nn, h = tab0.shape
    _, nchp, _ = e0p.shape
    ewp = nchp * CK
    nvec = h // 16
    mesh = plsc.VectorSubcoreMesh(core_axis_name="c", subcore_axis_name="s")

    @functools.partial(
        pl.kernel,
        out_type=jax.ShapeDtypeStruct((ne, h), jnp.float32),
        mesh=mesh,
        scratch_types=[
            pltpu.VMEM((nchp, CK), jnp.int32),
            pltpu.VMEM((nchp, CK), jnp.int32),
            pltpu.VMEM((CK, h), jnp.float32),
            pltpu.VMEM((CK, h), jnp.float32),
            pltpu.VMEM((CK, h), jnp.float32),
            pltpu.VMEM((CK, h), jnp.float32),
        ] + [pltpu.SemaphoreType.DMA] * 6,
    )
    def gk(t0, t1, e0h, e1h, og, idx0, idx1, r0a, r1a, r0b, r1b,
           sg0a, sg1a, sg0b, sg1b, soa, sob):
        wid = lax.axis_index("c") * NS + lax.axis_index("s")
        base = wid * ewp
        nch = jnp.minimum(ewp, ne - base) // CK
        pltpu.sync_copy(e0h.at[wid], idx0)
        pltpu.sync_copy(e1h.at[wid], idx1)

        def issue_g(j, r0, r1, s0, s1):
            pltpu.async_copy(t0.at[idx0.at[j]], r0, s0)
            pltpu.async_copy(t1.at[idx1.at[j]], r1, s1)

        def wait_g(j, r0, r1, s0, s1):
            pltpu.make_async_copy(t0.at[idx0.at[j]], r0, s0).wait()
            pltpu.make_async_copy(t1.at[idx1.at[j]], r1, s1).wait()

        def add_rows(r0, r1):
            def rows4(r4, carry):
                for rr in range(4):
                    r = r4 * 4 + rr
                    for c in range(nvec):
                        sl = pl.ds(c * 16, 16)
                        r0[r, sl] = r0[r, sl] + r1[r, sl]
                return carry
            lax.fori_loop(0, CK // 4, rows4, 0)

        def issue_o(j, r0, s):
            off = pl.multiple_of(base + j * CK, 8)
            pltpu.async_copy(r0, og.at[pl.ds(off, CK)], s)

        def wait_o(j, r0, s):
            off = pl.multiple_of(base + j * CK, 8)
            pltpu.make_async_copy(r0, og.at[pl.ds(off, CK)], s).wait()

        issue_g(0, r0a, r1a, sg0a, sg1a)
        issue_g(1, r0b, r1b, sg0b, sg1b)

        def body(i, carry):
            j = 2 * i
            wait_g(j, r0a, r1a, sg0a, sg1a)
            add_rows(r0a, r1a)
            issue_o(j, r0a, soa)

            @pl.when(j + 1 < nch)
            def _():
                wait_g(j + 1, r0b, r1b, sg0b, sg1b)
                add_rows(r0b, r1b)
                issue_o(j + 1, r0b, sob)

            @pl.when(j + 2 < nch)
            def _():
                wait_o(j, r0a, soa)
                issue_g(j + 2, r0a, r1a, sg0a, sg1a)

            @pl.when(j + 3 < nch)
            def _():
                wait_o(j + 1, r0b, sob)
                issue_g(j + 3, r0b, r1b, sg0b, sg1b)

            return carry

        lax.fori_loop(0, (nch + 1) // 2, body, 0)
        par = nch % 2
        wait_o(nch - 2 + par, r0a, soa)
        wait_o(nch - 1 - par, r0b, sob)

    return gk(tab0, tab1, e0p, e1p)


def _sc_scatter(x_edge_new, e0p, e1p, zeros):
    ne, h = x_edge_new.shape
    nn = zeros.shape[0]
    _, nchp, _ = e0p.shape
    ewp = nchp * CK
    npt = (nn // NS) // 8 * 8
    rem = nn - NS * npt
    mesh = plsc.VectorSubcoreMesh(core_axis_name="c", subcore_axis_name="s")

    @functools.partial(
        pl.kernel,
        out_type=jax.ShapeDtypeStruct((NC, nn, h), jnp.float32),
        mesh=mesh,
        scratch_types=[
            pltpu.VMEM_SHARED((nn, h), jnp.float32),
            pltpu.VMEM((1, CK), jnp.int32),
            pltpu.VMEM((1, CK), jnp.int32),
            pltpu.VMEM((1, CK), jnp.int32),
            pltpu.VMEM((1, CK), jnp.int32),
            pltpu.VMEM((CK, h), jnp.float32),
            pltpu.VMEM((CK, h), jnp.float32),
        ] + [pltpu.SemaphoreType.DMA] * 10,
    )
    def sk(xen, e0h, e1h, zz, oout, agg_sh, i0a, i1a, i0b, i1b, ra, rb,
           sla, slb, si0a, si1a, si0b, si1b, s0a, s1a, s0b, s1b):
        cid = lax.axis_index("c")
        sid = lax.axis_index("s")
        wid = cid * NS + sid
        base = wid * ewp
        nch = jnp.minimum(ewp, ne - base) // CK
        off_n = pl.multiple_of(sid * npt, 8)
        pltpu.sync_copy(zz.at[pl.ds(off_n, npt)],
                        agg_sh.at[pl.ds(off_n, npt)])
        if rem:
            @pl.when(sid == NS - 1)
            def _():
                pltpu.sync_copy(zz.at[pl.ds(NS * npt, rem)],
                                agg_sh.at[pl.ds(NS * npt, rem)])
        plsc.subcore_barrier()

        def issue_l(j, r, i0, i1, sr, s0, s1):
            off = pl.multiple_of(base + j * CK, 8)
            pltpu.async_copy(xen.at[pl.ds(off, CK)], r, sr)
            pltpu.async_copy(e0h.at[wid, j], i0.at[0], s0)
            pltpu.async_copy(e1h.at[wid, j], i1.at[0], s1)

        def wait_l(j, r, i0, i1, sr, s0, s1):
            off = pl.multiple_of(base + j * CK, 8)
            pltpu.make_async_copy(xen.at[pl.ds(off, CK)], r, sr).wait()
            pltpu.make_async_copy(e0h.at[wid, j], i0.at[0], s0).wait()
            pltpu.make_async_copy(e1h.at[wid, j], i1.at[0], s1).wait()

        def issue_s(r, i0, i1, s0, s1):
            pltpu.async_copy(r, agg_sh.at[i0.at[0]], s0, add=True)
            pltpu.async_copy(r, agg_sh.at[i1.at[0]], s1, add=True)

        def wait_s(r, i0, i1, s0, s1):
            pltpu.make_async_copy(r, agg_sh.at[i0.at[0]], s0).wait()
            pltpu.make_async_copy(r, agg_sh.at[i1.at[0]], s1).wait()

        issue_l(0, ra, i0a, i1a, sla, si0a, si1a)
        issue_l(1, rb, i0b, i1b, slb, si0b, si1b)

        def body(i, carry):
            j = 2 * i
            wait_l(j, ra, i0a, i1a, sla, si0a, si1a)
            issue_s(ra, i0a, i1a, s0a, s1a)

            @pl.when(j + 1 < nch)
            def _():
                wait_l(j + 1, rb, i0b, i1b, slb, si0b, si1b)
                issue_s(rb, i0b, i1b, s0b, s1b)

            @pl.when(j + 2 < nch)
            def _():
                wait_s(ra, i0a, i1a, s0a, s1a)
                issue_l(j + 2, ra, i0a, i1a, sla, si0a, si1a)

            @pl.when(j + 3 < nch)
            def _():
                wait_s(rb, i0b, i1b, s0b, s1b)
                issue_l(j + 3, rb, i0b, i1b, slb, si0b, si1b)

            return carry

        lax.fori_loop(0, (nch + 1) // 2, body, 0)
        wait_s(ra, i0a, i1a, s0a, s1a)
        wait_s(rb, i0b, i1b, s0b, s1b)
        plsc.subcore_barrier()
        pltpu.sync_copy(agg_sh.at[pl.ds(off_n, npt)],
                        oout.at[cid, pl.ds(off_n, npt)])
        if rem:
            @pl.when(sid == NS - 1)
            def _():
                pltpu.sync_copy(agg_sh.at[pl.ds(NS * npt, rem)],
                                oout.at[cid, pl.ds(NS * npt, rem)])

    return sk(x_edge_new, e0p, e1p, zeros)


def _bdot(x, w):
    return jnp.dot(x.astype(jnp.bfloat16), w.astype(jnp.bfloat16),
                   preferred_element_type=jnp.float32)


def _mlp_tail(x, w2, b2, w3, b3, w4, b4, g, b):
    x = jnp.maximum(_bdot(x, w2) + b2, 0.0)
    x = jnp.maximum(_bdot(x, w3) + b3, 0.0)
    x = _bdot(x, w4) + b4
    mu = jnp.mean(x, axis=-1, keepdims=True)
    xc = x - mu
    var = jnp.mean(xc * xc, axis=-1, keepdims=True)
    return xc * lax.rsqrt(var + 1e-5) * g + b


def _pq(x_node, w1a, w1b):
    nn, h = x_node.shape
    blk = 1000
    nspec = pl.BlockSpec((blk, h), lambda i: (i, 0))
    wspec = pl.BlockSpec((h, h), lambda i: (0, 0))

    def body(xr, war, wbr, pr, qr):
        pr[...] = jnp.dot(xr[...], war[...], preferred_element_type=jnp.float32)
        qr[...] = jnp.dot(xr[...], wbr[...], preferred_element_type=jnp.float32)

    return pl.pallas_call(
        body,
        grid=(nn // blk,),
        in_specs=[nspec, wspec, wspec],
        out_specs=[nspec, nspec],
        out_shape=[jax.ShapeDtypeStruct((nn, h), jnp.float32)] * 2,
    )(x_node, w1a, w1b)


def _edge_mlp(gsum, xe, w1c, b1, w2, b2, w3, b3, w4, b4, g, b):
    ne, h = xe.shape
    blk = 2560
    grid = ne // blk
    espec = pl.BlockSpec((blk, h), lambda i: (i, 0))
    wspec = pl.BlockSpec((h, h), lambda i: (0, 0))
    vspec = pl.BlockSpec((1, h), lambda i: (0, 0))

    def body(gsr, xer, w1cr, b1r, w2r, b2r, w3r, b3r,
             w4r, b4r, gr, br, xnr, oer):
        x = gsr[...] + _bdot(xer[...], w1cr[...]) + b1r[...]
        x = jnp.maximum(x, 0.0)
        xn = _mlp_tail(x, w2r[...], b2r[...], w3r[...], b3r[...],
                       w4r[...], b4r[...], gr[...], br[...])
        xnr[...] = xn
        oer[...] = xer[...] + xn

    return pl.pallas_call(
        body,
        grid=(grid,),
        in_specs=[espec, espec, wspec, vspec, wspec,
                  vspec, wspec, vspec, wspec, vspec, vspec, vspec],
        out_specs=[espec, espec],
        out_shape=[jax.ShapeDtypeStruct((ne, h), jnp.float32)] * 2,
    )(gsum, xe, w1c, b1, w2, b2, w3, b3, w4, b4, g, b)


def _node_mlp(x_node, aggp, u1a, u1b, c1, u2, c2, u3, c3, u4, c4, g, b):
    nn, h = x_node.shape
    blk = 1000
    grid = nn // blk
    nspec = pl.BlockSpec((blk, h), lambda i: (i, 0))
    aspec = pl.BlockSpec((NC, blk, h), lambda i: (0, i, 0))
    wspec = pl.BlockSpec((h, h), lambda i: (0, 0))
    vspec = pl.BlockSpec((1, h), lambda i: (0, 0))

    def body(xnr, apr, u1ar, u1br, c1r, u2r, c2r, u3r, c3r, u4r, c4r,
             gr, br, outr):
        agg = apr[0] + apr[1]
        x = _bdot(xnr[...], u1ar[...]) + _bdot(agg, u1br[...]) + c1r[...]
        x = jnp.maximum(x, 0.0)
        xn = _mlp_tail(x, u2r[...], c2r[...], u3r[...], c3r[...],
                       u4r[...], c4r[...], gr[...], br[...])
        outr[...] = xnr[...] + xn

    return pl.pallas_call(
        body,
        grid=(grid,),
        in_specs=[nspec, aspec, wspec, wspec, vspec, wspec, vspec, wspec,
                  vspec, wspec, vspec, vspec, vspec],
        out_specs=nspec,
        out_shape=jax.ShapeDtypeStruct((nn, h), jnp.float32),
    )(x_node, aggp, u1a, u1b, c1, u2, c2, u3, c3, u4, c4, g, b)


def kernel(x_node, x_edge, edge_index, eb_params, nb_params):
    eb_layers, (eg, ebb) = eb_params
    nb_layers, (ng, nbb) = nb_params
    (w1, b1), (w2, b2), (w3, b3), (w4, b4) = eb_layers
    (u1, c1), (u2, c2), (u3, c3), (u4, c4) = nb_layers
    nn, h = x_node.shape
    r2 = lambda a: a.reshape(1, h)

    ne = edge_index.shape[0]
    nchp = -(-ne // (NW * CK))
    pad = NW * nchp * CK - ne
    e0p = jnp.pad(edge_index[:, 0].astype(jnp.int32), (0, pad)).reshape(NW, nchp, CK)
    e1p = jnp.pad(edge_index[:, 1].astype(jnp.int32), (0, pad)).reshape(NW, nchp, CK)

    p, q = _pq(x_node, w1[:h], w1[h:2 * h])
    gsum = _sc_gather_add(p, q, e0p, e1p, ne)
    xen, out_edge = _edge_mlp(
        gsum, x_edge, w1[2 * h:], r2(b1),
        w2, r2(b2), w3, r2(b3), w4, r2(b4), r2(eg), r2(ebb))
    zeros = jnp.zeros((nn, h), jnp.float32)
    aggp = _sc_scatter(xen, e0p, e1p, zeros)
    out_node = _node_mlp(
        x_node, aggp, u1[:h], u1[h:], r2(c1),
        u2, r2(c2), u3, r2(c3), u4, r2(c4), r2(ng), r2(nbb))
    return (out_node, out_edge)


# speedup vs baseline: 1.1139x; 1.0298x over previous
import functools
import math

import jax
import jax.numpy as jnp
from jax import lax
from jax.experimental import pallas as pl
from jax.experimental.pallas import tpu as pltpu
from jax.experimental.pallas import tpu_sc as plsc

NC = 2
NS = 16
NW = NC * NS
CK = 128


def _sc_gather_add(tab0, tab1, e0p, e1p, ne):
    nn, h = tab0.shape
    _, nchp, _ = e0p.shape
    ewp = nchp * CK
    nvec = h // 16
    mesh = plsc.VectorSubcoreMesh(core_axis_name="c", subcore_axis_name="s")

    @functools.partial(
        pl.kernel,
        out_type=jax.ShapeDtypeStruct((ne, h), jnp.float32),
        mesh=mesh,
        scratch_types=[
            pltpu.VMEM((nchp, CK), jnp.int32),
            pltpu.VMEM((nchp, CK), jnp.int32),
            pltpu.VMEM((CK, h), jnp.float32),
            pltpu.VMEM((CK, h), jnp.float32),
            pltpu.VMEM((CK, h), jnp.float32),
            pltpu.VMEM((CK, h), jnp.float32),
        ] + [pltpu.SemaphoreType.DMA] * 6,
    )
    def gk(t0, t1, e0h, e1h, og, idx0, idx1, r0a, r1a, r0b, r1b,
           sg0a, sg1a, sg0b, sg1b, soa, sob):
        wid = lax.axis_index("c") * NS + lax.axis_index("s")
        base = wid * ewp
        nch = jnp.maximum(jnp.minimum(ewp, ne - base), 0) // CK
        pltpu.sync_copy(e0h.at[wid], idx0)
        pltpu.sync_copy(e1h.at[wid], idx1)

        def issue_g(j, r0, r1, s0, s1):
            pltpu.async_copy(t0.at[idx0.at[j]], r0, s0)
            pltpu.async_copy(t1.at[idx1.at[j]], r1, s1)

        def wait_g(j, r0, r1, s0, s1):
            pltpu.make_async_copy(t0.at[idx0.at[j]], r0, s0).wait()
            pltpu.make_async_copy(t1.at[idx1.at[j]], r1, s1).wait()

        def add_rows(r0, r1):
            def rows4(r4, carry):
                for rr in range(4):
                    r = r4 * 4 + rr
                    for c in range(nvec):
                        sl = pl.ds(c * 16, 16)
                        r0[r, sl] = r0[r, sl] + r1[r, sl]
                return carry
            lax.fori_loop(0, CK // 4, rows4, 0)

        def issue_o(j, r0, s):
            off = pl.multiple_of(base + j * CK, 8)
            pltpu.async_copy(r0, og.at[pl.ds(off, CK)], s)

        def wait_o(j, r0, s):
            off = pl.multiple_of(base + j * CK, 8)
            pltpu.make_async_copy(r0, og.at[pl.ds(off, CK)], s).wait()

        @pl.when(nch >= 1)
        def _():
            issue_g(0, r0a, r1a, sg0a, sg1a)

        @pl.when(nch >= 2)
        def _():
            issue_g(1, r0b, r1b, sg0b, sg1b)

        def body(i, carry):
            j = 2 * i
            wait_g(j, r0a, r1a, sg0a, sg1a)
            add_rows(r0a, r1a)
            issue_o(j, r0a, soa)

            @pl.when(j + 1 < nch)
            def _():
                wait_g(j + 1, r0b, r1b, sg0b, sg1b)
                add_rows(r0b, r1b)
                issue_o(j + 1, r0b, sob)

            @pl.when(j + 2 < nch)
            def _():
                wait_o(j, r0a, soa)
                issue_g(j + 2, r0a, r1a, sg0a, sg1a)

            @pl.when(j + 3 < nch)
            def _():
                wait_o(j + 1, r0b, sob)
                issue_g(j + 3, r0b, r1b, sg0b, sg1b)

            return carry

        lax.fori_loop(0, (nch + 1) // 2, body, 0)
        par = nch % 2

        @pl.when(nch >= 1)
        def _():
            wait_o(nch - 2 + par, r0a, soa)

        @pl.when(nch >= 2)
        def _():
            wait_o(nch - 1 - par, r0b, sob)

    return gk(tab0, tab1, e0p, e1p)


def _sc_scatter(x_edge_new, e0p, e1p, seed):
    ne, h = x_edge_new.shape
    nn = seed.shape[1]
    _, nchp, _ = e0p.shape
    ewp = nchp * CK
    npt = (nn // NS) // 8 * 8
    rem = nn - NS * npt
    mesh = plsc.VectorSubcoreMesh(core_axis_name="c", subcore_axis_name="s")

    @functools.partial(
        pl.kernel,
        out_type=jax.ShapeDtypeStruct((NC, nn, h), jnp.float32),
        mesh=mesh,
        scratch_types=[
            pltpu.VMEM_SHARED((nn, h), jnp.float32),
            pltpu.VMEM((1, CK), jnp.int32),
            pltpu.VMEM((1, CK), jnp.int32),
            pltpu.VMEM((1, CK), jnp.int32),
            pltpu.VMEM((1, CK), jnp.int32),
            pltpu.VMEM((CK, h), jnp.float32),
            pltpu.VMEM((CK, h), jnp.float32),
        ] + [pltpu.SemaphoreType.DMA] * 10,
    )
    def sk(xen, e0h, e1h, zz, oout, agg_sh, i0a, i1a, i0b, i1b, ra, rb,
           sla, slb, si0a, si1a, si0b, si1b, s0a, s1a, s0b, s1b):
        cid = lax.axis_index("c")
        sid = lax.axis_index("s")
        wid = cid * NS + sid
        base = wid * ewp
        nch = jnp.maximum(jnp.minimum(ewp, ne - base), 0) // CK
        off_n = pl.multiple_of(sid * npt, 8)
        pltpu.sync_copy(zz.at[cid, pl.ds(off_n, npt)],
                        agg_sh.at[pl.ds(off_n, npt)])
        if rem:
            @pl.when(sid == NS - 1)
            def _():
                pltpu.sync_copy(zz.at[cid, pl.ds(NS * npt, rem)],
                                agg_sh.at[pl.ds(NS * npt, rem)])
        plsc.subcore_barrier()

        def issue_l(j, r, i0, i1, sr, s0, s1):
            off = pl.multiple_of(base + j * CK, 8)
            pltpu.async_copy(xen.at[pl.ds(off, CK)], r, sr)
            pltpu.async_copy(e0h.at[wid, j], i0.at[0], s0)
            pltpu.async_copy(e1h.at[wid, j], i1.at[0], s1)

        def wait_l(j, r, i0, i1, sr, s0, s1):
            off = pl.multiple_of(base + j * CK, 8)
            pltpu.make_async_copy(xen.at[pl.ds(off, CK)], r, sr).wait()
            pltpu.make_async_copy(e0h.at[wid, j], i0.at[0], s0).wait()
            pltpu.make_async_copy(e1h.at[wid, j], i1.at[0], s1).wait()

        def issue_s(r, i0, i1, s0, s1):
            pltpu.async_copy(r, agg_sh.at[i0.at[0]], s0, add=True)
            pltpu.async_copy(r, agg_sh.at[i1.at[0]], s1, add=True)

        def wait_s(r, i0, i1, s0, s1):
            pltpu.make_async_copy(r, agg_sh.at[i0.at[0]], s0).wait()
            pltpu.make_async_copy(r, agg_sh.at[i1.at[0]], s1).wait()

        @pl.when(nch >= 1)
        def _():
            issue_l(0, ra, i0a, i1a, sla, si0a, si1a)

        @pl.when(nch >= 2)
        def _():
            issue_l(1, rb, i0b, i1b, slb, si0b, si1b)

        def body(i, carry):
            j = 2 * i
            wait_l(j, ra, i0a, i1a, sla, si0a, si1a)
            issue_s(ra, i0a, i1a, s0a, s1a)

            @pl.when(j + 1 < nch)
            def _():
                wait_l(j + 1, rb, i0b, i1b, slb, si0b, si1b)
                issue_s(rb, i0b, i1b, s0b, s1b)

            @pl.when(j + 2 < nch)
            def _():
                wait_s(ra, i0a, i1a, s0a, s1a)
                issue_l(j + 2, ra, i0a, i1a, sla, si0a, si1a)

            @pl.when(j + 3 < nch)
            def _():
                wait_s(rb, i0b, i1b, s0b, s1b)
                issue_l(j + 3, rb, i0b, i1b, slb, si0b, si1b)

            return carry

        lax.fori_loop(0, (nch + 1) // 2, body, 0)
        @pl.when(nch >= 1)
        def _():
            wait_s(ra, i0a, i1a, s0a, s1a)

        @pl.when(nch >= 2)
        def _():
            wait_s(rb, i0b, i1b, s0b, s1b)

        plsc.subcore_barrier()
        pltpu.sync_copy(agg_sh.at[pl.ds(off_n, npt)],
                        oout.at[cid, pl.ds(off_n, npt)])
        if rem:
            @pl.when(sid == NS - 1)
            def _():
                pltpu.sync_copy(agg_sh.at[pl.ds(NS * npt, rem)],
                                oout.at[cid, pl.ds(NS * npt, rem)])

    return sk(x_edge_new, e0p, e1p, seed)


def _bdot(x, w):
    return jnp.dot(x.astype(jnp.bfloat16), w.astype(jnp.bfloat16),
                   preferred_element_type=jnp.float32)


def _mlp_tail(x, w2, b2, w3, b3, w4, b4, g, b):
    x = jnp.maximum(_bdot(x, w2) + b2, 0.0)
    x = jnp.maximum(_bdot(x, w3) + b3, 0.0)
    x = _bdot(x, w4) + b4
    mu = jnp.mean(x, axis=-1, keepdims=True)
    xc = x - mu
    var = jnp.mean(xc * xc, axis=-1, keepdims=True)
    return xc * lax.rsqrt(var + 1e-5) * g + b


def _pq(x_node, w1a, w1b):
    nn, h = x_node.shape
    blk = 1000
    nspec = pl.BlockSpec((blk, h), lambda i: (i, 0))
    wspec = pl.BlockSpec((h, h), lambda i: (0, 0))

    def body(xr, war, wbr, pr, qr):
        pr[...] = jnp.dot(xr[...], war[...], preferred_element_type=jnp.float32)
        qr[...] = jnp.dot(xr[...], wbr[...], preferred_element_type=jnp.float32)

    return pl.pallas_call(
        body,
        grid=(nn // blk,),
        in_specs=[nspec, wspec, wspec],
        out_specs=[nspec, nspec],
        out_shape=[jax.ShapeDtypeStruct((nn, h), jnp.float32)] * 2,
    )(x_node, w1a, w1b)


def _edge_mlp(gsum, xe, w1c, b1, w2, b2, w3, b3, w4, b4, g, b, oe_seed, ob):
    neh, h = gsum.shape
    ne = xe.shape[0]
    blk = 2560
    grid = neh // blk
    lspec = pl.BlockSpec((blk, h), lambda i: (i, 0))
    ospec = pl.BlockSpec((blk, h), lambda i: (i + ob, 0))
    wspec = pl.BlockSpec((h, h), lambda i: (0, 0))
    vspec = pl.BlockSpec((1, h), lambda i: (0, 0))
    aspec = pl.BlockSpec(memory_space=pl.ANY)

    def body(gsr, xer, w1cr, b1r, w2r, b2r, w3r, b3r,
             w4r, b4r, gr, br, _seed, xnr, oer):
        x = gsr[...] + _bdot(xer[...], w1cr[...]) + b1r[...]
        x = jnp.maximum(x, 0.0)
        xn = _mlp_tail(x, w2r[...], b2r[...], w3r[...], b3r[...],
                       w4r[...], b4r[...], gr[...], br[...])
        xnr[...] = xn
        oer[...] = xer[...] + xn

    return pl.pallas_call(
        body,
        grid=(grid,),
        in_specs=[lspec, ospec, wspec, vspec, wspec, vspec, wspec, vspec,
                  wspec, vspec, vspec, vspec, aspec],
        out_specs=[lspec, ospec],
        out_shape=[jax.ShapeDtypeStruct((neh, h), jnp.float32),
                   jax.ShapeDtypeStruct((ne, h), jnp.float32)],
        input_output_aliases={12: 1},
    )(gsum, xe, w1c, b1, w2, b2, w3, b3, w4, b4, g, b, oe_seed)


def _node_mlp(x_node, aggp, u1a, u1b, c1, u2, c2, u3, c3, u4, c4, g, b):
    nn, h = x_node.shape
    blk = 1000
    grid = nn // blk
    nspec = pl.BlockSpec((blk, h), lambda i: (i, 0))
    aspec = pl.BlockSpec((NC, blk, h), lambda i: (0, i, 0))
    wspec = pl.BlockSpec((h, h), lambda i: (0, 0))
    vspec = pl.BlockSpec((1, h), lambda i: (0, 0))

    def body(xnr, apr, u1ar, u1br, c1r, u2r, c2r, u3r, c3r, u4r, c4r,
             gr, br, outr):
        agg = apr[0] + apr[1]
        x = _bdot(xnr[...], u1ar[...]) + _bdot(agg, u1br[...]) + c1r[...]
        x = jnp.maximum(x, 0.0)
        xn = _mlp_tail(x, u2r[...], c2r[...], u3r[...], c3r[...],
                       u4r[...], c4r[...], gr[...], br[...])
        outr[...] = xnr[...] + xn

    return pl.pallas_call(
        body,
        grid=(grid,),
        in_specs=[nspec, aspec, wspec, wspec, vspec, wspec, vspec, wspec,
                  vspec, wspec, vspec, vspec, vspec],
        out_specs=nspec,
        out_shape=jax.ShapeDtypeStruct((nn, h), jnp.float32),
    )(x_node, aggp, u1a, u1b, c1, u2, c2, u3, c3, u4, c4, g, b)


def kernel(x_node, x_edge, edge_index, eb_params, nb_params):
    eb_layers, (eg, ebb) = eb_params
    nb_layers, (ng, nbb) = nb_params
    (w1, b1), (w2, b2), (w3, b3), (w4, b4) = eb_layers
    (u1, c1), (u2, c2), (u3, c3), (u4, c4) = nb_layers
    nn, h = x_node.shape
    r2 = lambda a: a.reshape(1, h)

    ne = edge_index.shape[0]
    blk = 2560
    ew = NW * CK
    align = (NW * CK) * blk // math.gcd(NW * CK, blk)
    nea = (ne // 2) // align * align
    neb = ne - nea
    nchpa = nea // (NW * CK)
    nchpb = -(-neb // (NW * CK))
    padb = NW * nchpb * CK - neb
    e0 = edge_index[:, 0].astype(jnp.int32)
    e1 = edge_index[:, 1].astype(jnp.int32)
    e0pa = e0[:nea].reshape(NW, nchpa, CK)
    e1pa = e1[:nea].reshape(NW, nchpa, CK)
    e0pb = jnp.pad(e0[nea:], (0, padb)).reshape(NW, nchpb, CK)
    e1pb = jnp.pad(e1[nea:], (0, padb)).reshape(NW, nchpb, CK)

    p, q = _pq(x_node, w1[:h], w1[h:2 * h])
    ga = _sc_gather_add(p, q, e0pa, e1pa, nea)
    gb = _sc_gather_add(p, q, e0pb, e1pb, neb)
    oe_seed = jnp.zeros((ne, h), jnp.float32)
    xena, oe1 = _edge_mlp(
        ga, x_edge, w1[2 * h:], r2(b1), w2, r2(b2), w3, r2(b3), w4, r2(b4),
        r2(eg), r2(ebb), oe_seed, 0)
    xenb, out_edge = _edge_mlp(
        gb, x_edge, w1[2 * h:], r2(b1), w2, r2(b2), w3, r2(b3), w4, r2(b4),
        r2(eg), r2(ebb), oe1, nea // blk)
    seed0 = jnp.zeros((NC, nn, h), jnp.float32)
    pa = _sc_scatter(xena, e0pa, e1pa, seed0)
    pb = _sc_scatter(xenb, e0pb, e1pb, pa)
    out_node = _node_mlp(
        x_node, pb, u1[:h], u1[h:], r2(c1),
        u2, r2(c2), u3, r2(c3), u4, r2(c4), r2(ng), r2(nbb))
    return (out_node, out_edge)


# speedup vs baseline: 1.2707x; 1.1407x over previous
import functools
import math

import jax
import jax.numpy as jnp
from jax import lax
from jax.experimental import pallas as pl
from jax.experimental.pallas import tpu as pltpu
from jax.experimental.pallas import tpu_sc as plsc

NC = 2
NS = 16
NW = NC * NS
CK = 128


def _sc_gather_add(tab0, tab1, e0p, e1p, ne):
    nn, h = tab0.shape
    _, nchp, _ = e0p.shape
    ewp = nchp * CK
    nvec = h // 16
    mesh = plsc.VectorSubcoreMesh(core_axis_name="c", subcore_axis_name="s")

    @functools.partial(
        pl.kernel,
        out_type=jax.ShapeDtypeStruct((ne, h), jnp.float32),
        mesh=mesh,
        scratch_types=[
            pltpu.VMEM((nchp, CK), jnp.int32),
            pltpu.VMEM((nchp, CK), jnp.int32),
            pltpu.VMEM((CK, h), jnp.float32),
            pltpu.VMEM((CK, h), jnp.float32),
            pltpu.VMEM((CK, h), jnp.float32),
            pltpu.VMEM((CK, h), jnp.float32),
        ] + [pltpu.SemaphoreType.DMA] * 6,
    )
    def gk(t0, t1, e0h, e1h, og, idx0, idx1, r0a, r1a, r0b, r1b,
           sg0a, sg1a, sg0b, sg1b, soa, sob):
        wid = lax.axis_index("c") * NS + lax.axis_index("s")
        base = wid * ewp
        nch = jnp.maximum(jnp.minimum(ewp, ne - base), 0) // CK
        pltpu.sync_copy(e0h.at[wid], idx0)
        pltpu.sync_copy(e1h.at[wid], idx1)

        def issue_g(j, r0, r1, s0, s1):
            pltpu.async_copy(t0.at[idx0.at[j]], r0, s0)
            pltpu.async_copy(t1.at[idx1.at[j]], r1, s1)

        def wait_g(j, r0, r1, s0, s1):
            pltpu.make_async_copy(t0.at[idx0.at[j]], r0, s0).wait()
            pltpu.make_async_copy(t1.at[idx1.at[j]], r1, s1).wait()

        def add_rows(r0, r1):
            def rows4(r4, carry):
                for rr in range(4):
                    r = r4 * 4 + rr
                    for c in range(nvec):
                        sl = pl.ds(c * 16, 16)
                        r0[r, sl] = r0[r, sl] + r1[r, sl]
                return carry
            lax.fori_loop(0, CK // 4, rows4, 0)

        def issue_o(j, r0, s):
            off = pl.multiple_of(base + j * CK, 8)
            pltpu.async_copy(r0, og.at[pl.ds(off, CK)], s)

        def wait_o(j, r0, s):
            off = pl.multiple_of(base + j * CK, 8)
            pltpu.make_async_copy(r0, og.at[pl.ds(off, CK)], s).wait()

        @pl.when(nch >= 1)
        def _():
            issue_g(0, r0a, r1a, sg0a, sg1a)

        @pl.when(nch >= 2)
        def _():
            issue_g(1, r0b, r1b, sg0b, sg1b)

        def body(i, carry):
            j = 2 * i
            wait_g(j, r0a, r1a, sg0a, sg1a)
            add_rows(r0a, r1a)
            issue_o(j, r0a, soa)

            @pl.when(j + 1 < nch)
            def _():
                wait_g(j + 1, r0b, r1b, sg0b, sg1b)
                add_rows(r0b, r1b)
                issue_o(j + 1, r0b, sob)

            @pl.when(j + 2 < nch)
            def _():
                wait_o(j, r0a, soa)
                issue_g(j + 2, r0a, r1a, sg0a, sg1a)

            @pl.when(j + 3 < nch)
            def _():
                wait_o(j + 1, r0b, sob)
                issue_g(j + 3, r0b, r1b, sg0b, sg1b)

            return carry

        lax.fori_loop(0, (nch + 1) // 2, body, 0)
        par = nch % 2

        @pl.when(nch >= 1)
        def _():
            wait_o(nch - 2 + par, r0a, soa)

        @pl.when(nch >= 2)
        def _():
            wait_o(nch - 1 - par, r0b, sob)

    return gk(tab0, tab1, e0p, e1p)


def _sc_scatter(x_edge_new, e0p, e1p, seed):
    ne, h = x_edge_new.shape
    nn = seed.shape[1]
    _, nchp, _ = e0p.shape
    ewp = nchp * CK
    npt = (nn // NS) // 8 * 8
    rem = nn - NS * npt
    mesh = plsc.VectorSubcoreMesh(core_axis_name="c", subcore_axis_name="s")

    @functools.partial(
        pl.kernel,
        out_type=jax.ShapeDtypeStruct((NC, nn, h), jnp.float32),
        mesh=mesh,
        scratch_types=[
            pltpu.VMEM_SHARED((nn, h), jnp.float32),
            pltpu.VMEM((1, CK), jnp.int32),
            pltpu.VMEM((1, CK), jnp.int32),
            pltpu.VMEM((1, CK), jnp.int32),
            pltpu.VMEM((1, CK), jnp.int32),
            pltpu.VMEM((CK, h), jnp.float32),
            pltpu.VMEM((CK, h), jnp.float32),
        ] + [pltpu.SemaphoreType.DMA] * 10,
    )
    def sk(xen, e0h, e1h, zz, oout, agg_sh, i0a, i1a, i0b, i1b, ra, rb,
           sla, slb, si0a, si1a, si0b, si1b, s0a, s1a, s0b, s1b):
        cid = lax.axis_index("c")
        sid = lax.axis_index("s")
        wid = cid * NS + sid
        base = wid * ewp
        nch = jnp.maximum(jnp.minimum(ewp, ne - base), 0) // CK
        off_n = pl.multiple_of(sid * npt, 8)
        pltpu.sync_copy(zz.at[cid, pl.ds(off_n, npt)],
                        agg_sh.at[pl.ds(off_n, npt)])
        if rem:
            @pl.when(sid == NS - 1)
            def _():
                pltpu.sync_copy(zz.at[cid, pl.ds(NS * npt, rem)],
                                agg_sh.at[pl.ds(NS * npt, rem)])
        plsc.subcore_barrier()

        def issue_l(j, r, i0, i1, sr, s0, s1):
            off = pl.multiple_of(base + j * CK, 8)
            pltpu.async_copy(xen.at[pl.ds(off, CK)], r, sr)
            pltpu.async_copy(e0h.at[wid, j], i0.at[0], s0)
            pltpu.async_copy(e1h.at[wid, j], i1.at[0], s1)

        def wait_l(j, r, i0, i1, sr, s0, s1):
            off = pl.multiple_of(base + j * CK, 8)
            pltpu.make_async_copy(xen.at[pl.ds(off, CK)], r, sr).wait()
            pltpu.make_async_copy(e0h.at[wid, j], i0.at[0], s0).wait()
            pltpu.make_async_copy(e1h.at[wid, j], i1.at[0], s1).wait()

        def issue_s(r, i0, i1, s0, s1):
            pltpu.async_copy(r, agg_sh.at[i0.at[0]], s0, add=True)
            pltpu.async_copy(r, agg_sh.at[i1.at[0]], s1, add=True)

        def wait_s(r, i0, i1, s0, s1):
            pltpu.make_async_copy(r, agg_sh.at[i0.at[0]], s0).wait()
            pltpu.make_async_copy(r, agg_sh.at[i1.at[0]], s1).wait()

        @pl.when(nch >= 1)
        def _():
            issue_l(0, ra, i0a, i1a, sla, si0a, si1a)

        @pl.when(nch >= 2)
        def _():
            issue_l(1, rb, i0b, i1b, slb, si0b, si1b)

        def body(i, carry):
            j = 2 * i
            wait_l(j, ra, i0a, i1a, sla, si0a, si1a)
            issue_s(ra, i0a, i1a, s0a, s1a)

            @pl.when(j + 1 < nch)
            def _():
                wait_l(j + 1, rb, i0b, i1b, slb, si0b, si1b)
                issue_s(rb, i0b, i1b, s0b, s1b)

            @pl.when(j + 2 < nch)
            def _():
                wait_s(ra, i0a, i1a, s0a, s1a)
                issue_l(j + 2, ra, i0a, i1a, sla, si0a, si1a)

            @pl.when(j + 3 < nch)
            def _():
                wait_s(rb, i0b, i1b, s0b, s1b)
                issue_l(j + 3, rb, i0b, i1b, slb, si0b, si1b)

            return carry

        lax.fori_loop(0, (nch + 1) // 2, body, 0)
        @pl.when(nch >= 1)
        def _():
            wait_s(ra, i0a, i1a, s0a, s1a)

        @pl.when(nch >= 2)
        def _():
            wait_s(rb, i0b, i1b, s0b, s1b)

        plsc.subcore_barrier()
        pltpu.sync_copy(agg_sh.at[pl.ds(off_n, npt)],
                        oout.at[cid, pl.ds(off_n, npt)])
        if rem:
            @pl.when(sid == NS - 1)
            def _():
                pltpu.sync_copy(agg_sh.at[pl.ds(NS * npt, rem)],
                                oout.at[cid, pl.ds(NS * npt, rem)])

    return sk(x_edge_new, e0p, e1p, seed)


def _bdot(x, w):
    return jnp.dot(x.astype(jnp.bfloat16), w.astype(jnp.bfloat16),
                   preferred_element_type=jnp.float32)


def _mlp_tail(x, w2, b2, w3, b3, w4, b4, g, b):
    x = jnp.maximum(_bdot(x, w2) + b2, 0.0)
    x = jnp.maximum(_bdot(x, w3) + b3, 0.0)
    x = _bdot(x, w4) + b4
    mu = jnp.mean(x, axis=-1, keepdims=True)
    xc = x - mu
    var = jnp.mean(xc * xc, axis=-1, keepdims=True)
    return xc * lax.rsqrt(var + 1e-5) * g + b


def _pq(x_node, w1a, w1b):
    nn, h = x_node.shape
    blk = 1000
    nspec = pl.BlockSpec((blk, h), lambda i: (i, 0))
    wspec = pl.BlockSpec((h, h), lambda i: (0, 0))

    def body(xr, war, wbr, pr, qr):
        pr[...] = jnp.dot(xr[...], war[...], preferred_element_type=jnp.float32)
        qr[...] = jnp.dot(xr[...], wbr[...], preferred_element_type=jnp.float32)

    return pl.pallas_call(
        body,
        grid=(nn // blk,),
        in_specs=[nspec, wspec, wspec],
        out_specs=[nspec, nspec],
        out_shape=[jax.ShapeDtypeStruct((nn, h), jnp.float32)] * 2,
    )(x_node, w1a, w1b)


def _alloc(ne, h):
    def body(outr):
        outr[...] = jnp.zeros_like(outr)

    return pl.pallas_call(
        body,
        grid=(1,),
        out_specs=pl.BlockSpec((8, h), lambda i: (0, 0)),
        out_shape=jax.ShapeDtypeStruct((ne, h), jnp.float32),
    )()


def _edge_mlp(gsum, xe, w1c, b1, w2, b2, w3, b3, w4, b4, g, b, oe_seed, ob):
    neh, h = gsum.shape
    ne = xe.shape[0]
    blk = 2560
    grid = neh // blk
    lspec = pl.BlockSpec((blk, h), lambda i: (i, 0))
    ospec = pl.BlockSpec((blk, h), lambda i: (i + ob, 0))
    wspec = pl.BlockSpec((h, h), lambda i: (0, 0))
    vspec = pl.BlockSpec((1, h), lambda i: (0, 0))
    aspec = pl.BlockSpec(memory_space=pl.ANY)

    def body(gsr, xer, w1cr, b1r, w2r, b2r, w3r, b3r,
             w4r, b4r, gr, br, _seed, xnr, oer):
        x = gsr[...] + _bdot(xer[...], w1cr[...]) + b1r[...]
        x = jnp.maximum(x, 0.0)
        xn = _mlp_tail(x, w2r[...], b2r[...], w3r[...], b3r[...],
                       w4r[...], b4r[...], gr[...], br[...])
        xnr[...] = xn
        oer[...] = xer[...] + xn

    return pl.pallas_call(
        body,
        grid=(grid,),
        in_specs=[lspec, ospec, wspec, vspec, wspec, vspec, wspec, vspec,
                  wspec, vspec, vspec, vspec, aspec],
        out_specs=[lspec, ospec],
        out_shape=[jax.ShapeDtypeStruct((neh, h), jnp.float32),
                   jax.ShapeDtypeStruct((ne, h), jnp.float32)],
        input_output_aliases={12: 1},
    )(gsum, xe, w1c, b1, w2, b2, w3, b3, w4, b4, g, b, oe_seed)


def _node_mlp(x_node, aggp, u1a, u1b, c1, u2, c2, u3, c3, u4, c4, g, b):
    nn, h = x_node.shape
    blk = 1000
    grid = nn // blk
    nspec = pl.BlockSpec((blk, h), lambda i: (i, 0))
    aspec = pl.BlockSpec((NC, blk, h), lambda i: (0, i, 0))
    wspec = pl.BlockSpec((h, h), lambda i: (0, 0))
    vspec = pl.BlockSpec((1, h), lambda i: (0, 0))

    def body(xnr, apr, u1ar, u1br, c1r, u2r, c2r, u3r, c3r, u4r, c4r,
             gr, br, outr):
        agg = apr[0] + apr[1]
        x = _bdot(xnr[...], u1ar[...]) + _bdot(agg, u1br[...]) + c1r[...]
        x = jnp.maximum(x, 0.0)
        xn = _mlp_tail(x, u2r[...], c2r[...], u3r[...], c3r[...],
                       u4r[...], c4r[...], gr[...], br[...])
        outr[...] = xnr[...] + xn

    return pl.pallas_call(
        body,
        grid=(grid,),
        in_specs=[nspec, aspec, wspec, wspec, vspec, wspec, vspec, wspec,
                  vspec, wspec, vspec, vspec, vspec],
        out_specs=nspec,
        out_shape=jax.ShapeDtypeStruct((nn, h), jnp.float32),
    )(x_node, aggp, u1a, u1b, c1, u2, c2, u3, c3, u4, c4, g, b)


def kernel(x_node, x_edge, edge_index, eb_params, nb_params):
    eb_layers, (eg, ebb) = eb_params
    nb_layers, (ng, nbb) = nb_params
    (w1, b1), (w2, b2), (w3, b3), (w4, b4) = eb_layers
    (u1, c1), (u2, c2), (u3, c3), (u4, c4) = nb_layers
    nn, h = x_node.shape
    r2 = lambda a: a.reshape(1, h)

    ne = edge_index.shape[0]
    blk = 2560
    ew = NW * CK
    align = (NW * CK) * blk // math.gcd(NW * CK, blk)
    nea = (ne + align) // (2 * align) * align
    neb = ne - nea
    nchpa = nea // (NW * CK)
    nchpb = -(-neb // (NW * CK))
    padb = NW * nchpb * CK - neb
    e0 = edge_index[:, 0].astype(jnp.int32)
    e1 = edge_index[:, 1].astype(jnp.int32)
    e0pa = e0[:nea].reshape(NW, nchpa, CK)
    e1pa = e1[:nea].reshape(NW, nchpa, CK)
    e0pb = jnp.pad(e0[nea:], (0, padb)).reshape(NW, nchpb, CK)
    e1pb = jnp.pad(e1[nea:], (0, padb)).reshape(NW, nchpb, CK)

    p, q = _pq(x_node, w1[:h], w1[h:2 * h])
    ga = _sc_gather_add(p, q, e0pa, e1pa, nea)
    gb = _sc_gather_add(p, q, e0pb, e1pb, neb)
    oe_seed = _alloc(ne, h)
    xena, oe1 = _edge_mlp(
        ga, x_edge, w1[2 * h:], r2(b1), w2, r2(b2), w3, r2(b3), w4, r2(b4),
        r2(eg), r2(ebb), oe_seed, 0)
    xenb, out_edge = _edge_mlp(
        gb, x_edge, w1[2 * h:], r2(b1), w2, r2(b2), w3, r2(b3), w4, r2(b4),
        r2(eg), r2(ebb), oe1, nea // blk)
    seed0 = jnp.zeros((NC, nn, h), jnp.float32)
    pa = _sc_scatter(xena, e0pa, e1pa, seed0)
    pb = _sc_scatter(xenb, e0pb, e1pb, pa)
    out_node = _node_mlp(
        x_node, pb, u1[:h], u1[h:], r2(c1),
        u2, r2(c2), u3, r2(c3), u4, r2(c4), r2(ng), r2(nbb))
    return (out_node, out_edge)
